# Initial kernel scaffold; baseline (speedup 1.0000x reference)
import jax
import jax.numpy as jnp
from jax.experimental import pallas as pl


def kernel(x, edge_index, W1, as1, ad1, b1, W2, as2, ad2, b2, W3, as3, ad3, b3):
    raise NotImplementedError("write your pallas kernel here")


# speedup vs baseline: 3.3515x; 3.3515x over previous
import functools

import jax
import jax.numpy as jnp
from jax.experimental import pallas as pl

_NEG_SLOPE = 0.2
_EPS = 1e-16

_BN = 2000
_BE = 5000


def _leaky(v):
    return jnp.where(v >= 0, v, _NEG_SLOPE * v)


def _node_kernel(x_ref, w_ref, asb_ref, adb_ref, b_ref, h_ref, asrc_ref,
                 adst_ref, *, apply_act):
    x = x_ref[...]
    if apply_act:
        x = _leaky(x + b_ref[...])
    h = jnp.dot(x, w_ref[...], preferred_element_type=jnp.float32)
    h_ref[...] = h
    asrc_ref[...] = jnp.dot(h, asb_ref[...], preferred_element_type=jnp.float32)
    adst_ref[...] = jnp.dot(h, adb_ref[...], preferred_element_type=jnp.float32)


def _node_stage(x, b_prev, W, asb, adb, apply_act):
    n, inc = x.shape
    outc = W.shape[1]
    ha = asb.shape[1]
    kern = functools.partial(_node_kernel, apply_act=apply_act)
    return pl.pallas_call(
        kern,
        grid=(n // _BN,),
        in_specs=[
            pl.BlockSpec((_BN, inc), lambda i: (i, 0)),
            pl.BlockSpec((inc, outc), lambda i: (0, 0)),
            pl.BlockSpec((outc, ha), lambda i: (0, 0)),
            pl.BlockSpec((outc, ha), lambda i: (0, 0)),
            pl.BlockSpec((1, inc), lambda i: (0, 0)),
        ],
        out_specs=[
            pl.BlockSpec((_BN, outc), lambda i: (i, 0)),
            pl.BlockSpec((_BN, ha), lambda i: (i, 0)),
            pl.BlockSpec((_BN, ha), lambda i: (i, 0)),
        ],
        out_shape=[
            jax.ShapeDtypeStruct((n, outc), jnp.float32),
            jax.ShapeDtypeStruct((n, ha), jnp.float32),
            jax.ShapeDtypeStruct((n, ha), jnp.float32),
        ],
    )(x, W, asb, adb, b_prev)


def _edge_e_kernel(a_ref, b_ref, o_ref):
    o_ref[...] = _leaky(a_ref[...] + b_ref[...])


def _edge_exp_kernel(e_ref, m_ref, o_ref):
    o_ref[...] = jnp.exp(e_ref[...] - m_ref[...])


def _ew2_call(f, a, b):
    e, c = a.shape
    return pl.pallas_call(
        f,
        grid=(e // _BE,),
        in_specs=[
            pl.BlockSpec((_BE, c), lambda i: (i, 0)),
            pl.BlockSpec((_BE, c), lambda i: (i, 0)),
        ],
        out_specs=pl.BlockSpec((_BE, c), lambda i: (i, 0)),
        out_shape=jax.ShapeDtypeStruct((e, c), jnp.float32),
    )(a, b)


def _edge_msg_kernel(ee_ref, d_ref, hg_ref, s_ref, o_ref):
    alpha = ee_ref[...] / (d_ref[...] + _EPS)
    wide = jnp.dot(alpha, s_ref[...], preferred_element_type=jnp.float32)
    o_ref[...] = hg_ref[...] * wide


def _msg_call(ee, dg, hg, S):
    e, ha = ee.shape
    hc = hg.shape[1]
    return pl.pallas_call(
        _edge_msg_kernel,
        grid=(e // _BE,),
        in_specs=[
            pl.BlockSpec((_BE, ha), lambda i: (i, 0)),
            pl.BlockSpec((_BE, ha), lambda i: (i, 0)),
            pl.BlockSpec((_BE, hc), lambda i: (i, 0)),
            pl.BlockSpec((ha, hc), lambda i: (0, 0)),
        ],
        out_specs=pl.BlockSpec((_BE, hc), lambda i: (i, 0)),
        out_shape=jax.ShapeDtypeStruct((e, hc), jnp.float32),
    )(ee, dg, hg, S)


def _bias_kernel(x_ref, b_ref, o_ref):
    o_ref[...] = x_ref[...] + b_ref[...]


def _bias_call(x, b):
    n, c = x.shape
    return pl.pallas_call(
        _bias_kernel,
        grid=(n // _BN,),
        in_specs=[
            pl.BlockSpec((_BN, c), lambda i: (i, 0)),
            pl.BlockSpec((1, c), lambda i: (0, 0)),
        ],
        out_specs=pl.BlockSpec((_BN, c), lambda i: (i, 0)),
        out_shape=jax.ShapeDtypeStruct((n, c), jnp.float32),
    )(x, b)


def _blockdiag(a):
    hh, cc = a.shape
    rows = (jnp.arange(hh)[:, None] * cc + jnp.arange(cc)[None, :]).reshape(-1)
    cols = jnp.repeat(jnp.arange(hh), cc)
    return jnp.zeros((hh * cc, hh), a.dtype).at[rows, cols].set(a.reshape(-1))


def _gat_layer(x, b_prev, apply_act, src, dst, W, a_s, a_d, n):
    heads, outc = a_s.shape
    asb = _blockdiag(a_s)
    adb = _blockdiag(a_d)
    S = _blockdiag(jnp.ones((heads, outc), jnp.float32)).T
    h, asrc, adst = _node_stage(x, b_prev, W, asb, adb, apply_act)
    e = _ew2_call(_edge_e_kernel, asrc[src], adst[dst])
    emax = jax.ops.segment_max(e, dst, num_segments=n)
    ee = _ew2_call(_edge_exp_kernel, e, emax[dst])
    denom = jax.ops.segment_sum(ee, dst, num_segments=n)
    msg = _msg_call(ee, denom[dst], h[src], S)
    return jax.ops.segment_sum(msg, dst, num_segments=n)


def kernel(x, edge_index, W1, as1, ad1, b1, W2, as2, ad2, b2, W3, as3, ad3,
           b3):
    n = x.shape[0]
    loop = jnp.arange(n, dtype=edge_index.dtype)
    src = jnp.concatenate([edge_index[0], loop])
    dst = jnp.concatenate([edge_index[1], loop])

    z6 = jnp.zeros((1, x.shape[1]), jnp.float32)
    agg1 = _gat_layer(x, z6, False, src, dst, W1, as1, ad1, n)
    agg2 = _gat_layer(agg1, b1[None, :], True, src, dst, W2, as2, ad2, n)
    agg3 = _gat_layer(agg2, b2[None, :], True, src, dst, W3, as3, ad3, n)
    return _bias_call(agg3, b3[None, :])
